# Initial kernel scaffold; baseline (speedup 1.0000x reference)
import jax
import jax.numpy as jnp
from jax.experimental import pallas as pl


def kernel(A, F, W1, b1, W2, b2, log_lams):
    raise NotImplementedError("write your pallas kernel here")


# speedup vs baseline: 1.0802x; 1.0802x over previous
import jax
import jax.numpy as jnp
from jax.experimental import pallas as pl
from jax.experimental.pallas import tpu as pltpu

N = 4096
D_IN = 256
H = 128
C = 32
K = 4
LAM_HAT = 0.9
A_SCAD = 3.7
EPS = 1e-8

BI = 256
BJ = 512
BP = 256


def _prep_kernel(A_ref, F_ref, W1_ref, b1_ref, W2_ref, b2_ref,
                 F0_ref, Dd_ref, dinv_ref):
    a = A_ref[...]
    dd = jnp.sum(a, axis=1, keepdims=True) + 1.0
    Dd_ref[...] = dd
    dinv_ref[...] = jax.lax.rsqrt(dd)
    h = jnp.maximum(
        jnp.dot(F_ref[...], W1_ref[...], preferred_element_type=jnp.float32)
        + b1_ref[...], 0.0)
    F0_ref[...] = (jnp.dot(h, W2_ref[...], preferred_element_type=jnp.float32)
                   + b2_ref[...])


def _iter_kernel(lam_ref, A_ref, Fc_ref, dinv_ref, Dd_ref, F0_ref,
                 out_ref, S_acc, P_acc):
    i = pl.program_id(0)
    j = pl.program_id(1)
    nj = pl.num_programs(1)

    lam_k = lam_ref[0]
    lam = 1.0 / LAM_HAT - 1.0
    alam = A_SCAD * lam_k
    inv_c = 1.0 / (2.0 * (A_SCAD - 1.0) * lam_k)

    dv_i = dinv_ref[pl.ds(i * BI, BI), :]
    xni = Fc_ref[pl.ds(i * BI, BI), :] * dv_i
    dv_j = dinv_ref[pl.ds(j * BJ, BJ), :]
    xnj = Fc_ref[pl.ds(j * BJ, BJ), :] * dv_j

    sqi = jnp.sum(xni * xni, axis=1, keepdims=True)
    sqj = jnp.sum(xnj * xnj, axis=1, keepdims=True).T

    g = jax.lax.dot_general(xni, xnj, (((1,), (1,)), ((), ())),
                            preferred_element_type=jnp.float32)
    z = jnp.maximum(sqi + sqj - 2.0 * g, 0.0)
    r = jax.lax.rsqrt(jnp.maximum(z, EPS * EPS))
    y = z * r
    t = jnp.maximum(jnp.minimum((alam - y) * inv_c, 0.5), 0.0)
    w = t * r

    row = i * BI + jax.lax.broadcasted_iota(jnp.int32, (BI, BJ), 0)
    col = j * BJ + jax.lax.broadcasted_iota(jnp.int32, (BI, BJ), 1)
    w = jnp.where(row == col, 0.0, w)

    wa = w * A_ref[...]
    s_part = jnp.sum(wa, axis=1, keepdims=True)
    p_part = jax.lax.dot_general(wa, xnj, (((1,), (0,)), ((), ())),
                                 preferred_element_type=jnp.float32)

    @pl.when(j == 0)
    def _():
        S_acc[...] = s_part
        P_acc[...] = p_part

    @pl.when(j > 0)
    def _():
        S_acc[...] += s_part
        P_acc[...] += p_part

    @pl.when(j == nj - 1)
    def _():
        q = S_acc[...] / Dd_ref[...] + lam
        out_ref[...] = (dv_i * P_acc[...] + lam * F0_ref[...]) / q


def _prep_call(A, F, W1, b1, W2, b2):
    return pl.pallas_call(
        _prep_kernel,
        grid=(N // BP,),
        in_specs=[
            pl.BlockSpec((BP, N), lambda i: (i, 0)),
            pl.BlockSpec((BP, D_IN), lambda i: (i, 0)),
            pl.BlockSpec((D_IN, H), lambda i: (0, 0)),
            pl.BlockSpec((1, H), lambda i: (0, 0)),
            pl.BlockSpec((H, C), lambda i: (0, 0)),
            pl.BlockSpec((1, C), lambda i: (0, 0)),
        ],
        out_specs=[
            pl.BlockSpec((BP, C), lambda i: (i, 0)),
            pl.BlockSpec((BP, 1), lambda i: (i, 0)),
            pl.BlockSpec((BP, 1), lambda i: (i, 0)),
        ],
        out_shape=[
            jax.ShapeDtypeStruct((N, C), jnp.float32),
            jax.ShapeDtypeStruct((N, 1), jnp.float32),
            jax.ShapeDtypeStruct((N, 1), jnp.float32),
        ],
        compiler_params=pltpu.CompilerParams(
            dimension_semantics=("arbitrary",)),
    )(A, F, W1, b1, W2, b2)


def _iter_call(lam_k, A, Fc, dinv, Dd, F0):
    return pl.pallas_call(
        _iter_kernel,
        grid=(N // BI, N // BJ),
        in_specs=[
            pl.BlockSpec(memory_space=pltpu.SMEM),
            pl.BlockSpec((BI, BJ), lambda i, j: (i, j)),
            pl.BlockSpec((N, C), lambda i, j: (0, 0)),
            pl.BlockSpec((N, 1), lambda i, j: (0, 0)),
            pl.BlockSpec((BI, 1), lambda i, j: (i, 0)),
            pl.BlockSpec((BI, C), lambda i, j: (i, 0)),
        ],
        out_specs=pl.BlockSpec((BI, C), lambda i, j: (i, 0)),
        out_shape=jax.ShapeDtypeStruct((N, C), jnp.float32),
        scratch_shapes=[
            pltpu.VMEM((BI, 1), jnp.float32),
            pltpu.VMEM((BI, C), jnp.float32),
        ],
        compiler_params=pltpu.CompilerParams(
            dimension_semantics=("parallel", "arbitrary")),
    )(lam_k, A, Fc, dinv, Dd, F0)


def kernel(A, F, W1, b1, W2, b2, log_lams):
    F0, Dd, dinv = _prep_call(A, F, W1, b1.reshape(1, H), W2, b2.reshape(1, C))
    lams = jnp.exp(log_lams)
    Fc = F0
    for k in range(K):
        Fc = _iter_call(lams[k].reshape(1), A, Fc, dinv, Dd, F0)
    return Fc
